# baseline (speedup 1.0000x reference)
import functools

import jax
import jax.numpy as jnp
from jax import lax
from jax.experimental import pallas as pl
from jax.experimental.pallas import tpu as pltpu
from jax.experimental.pallas import tpu_sc as plsc

N = 400
B = 16
HID = 16
D = N * (N - 1) // 2
NC, NS = 2, 16
NG = N // 16
NPAIR = (NG - 1) // 2


def _unpack_body(data_hbm, adj_hbm, data_v, rowbuf_v):
    w = lax.axis_index("c")
    b = lax.axis_index("s")
    pltpu.sync_copy(data_hbm.at[b], data_v)
    lanes = lax.iota(jnp.int32, 16)
    jvs = [lanes + cc * 16 for cc in range(NG)]

    def do_group(g):
        base_i = g * 16
        ups = []
        for r in range(16):
            i = base_i + r
            oi = 399 * i - lax.shift_right_logical(i * (i - 1), 1)
            ups.append(oi - i - 1)
        for cc in range(NG):
            @pl.when(cc >= g)
            def _():
                idxs = [jvs[cc] + ups[r] for r in range(16)]
                if cc == 0:
                    idxs = [jnp.maximum(ix, 0) for ix in idxs]
                vals = [plsc.load_gather(data_v, [ix]) for ix in idxs]
                for r in range(16):
                    rowbuf_v[r, pl.ds(cc * 16, 16)] = vals[r]
        pltpu.sync_copy(rowbuf_v, adj_hbm.at[b, pl.ds(base_i, 16)])

    def pair_loop(p, carry):
        ga = p + NPAIR // 2 * w
        do_group(ga)
        do_group(NG - 1 - ga)
        return carry

    lax.fori_loop(0, NPAIR // 2, pair_loop, 0)

    mid_base = (NG // 2) * 16 + 8 * w
    mids = []
    for r in range(8):
        i = mid_base + r
        oi = 399 * i - lax.shift_right_logical(i * (i - 1), 1)
        mids.append(oi - i - 1)
    for cc in range(NG // 2, NG):
        vals = [plsc.load_gather(data_v, [jvs[cc] + mids[r]]) for r in range(8)]
        for r in range(8):
            rowbuf_v[r, pl.ds(cc * 16, 16)] = vals[r]
    pltpu.sync_copy(rowbuf_v.at[pl.ds(0, 8)],
                    adj_hbm.at[b, pl.ds(mid_base, 8)])


@functools.cache
def _build_unpack():
    return pl.kernel(
        _unpack_body,
        out_type=jax.ShapeDtypeStruct((B, N, N), jnp.float32),
        mesh=plsc.VectorSubcoreMesh(
            core_axis_name="c", subcore_axis_name="s",
            num_cores=NC, num_subcores=NS,
        ),
        scratch_types=[
            pltpu.VMEM((D,), jnp.float32),
            pltpu.VMEM((16, N), jnp.float32),
        ],
        compiler_params=pltpu.CompilerParams(needs_layout_passes=False),
    )


def _gcn_body(adj_ref, W0_ref, b0_ref, g0_ref, be0_ref,
              W1_ref, b1_ref, g1_ref, be1_ref, out_ref):
    n_tot = jnp.float32(B * N)

    def bn_coeffs(gamma, beta):
        ssum = jnp.zeros((1, HID), jnp.float32)
        for b in range(B):
            ssum = ssum + jnp.sum(out_ref[b * N:(b + 1) * N, :],
                                  axis=0, keepdims=True)
        mu = ssum / n_tot
        ssq = jnp.zeros((1, HID), jnp.float32)
        for b in range(B):
            dlt = out_ref[b * N:(b + 1) * N, :] - mu
            ssq = ssq + jnp.sum(dlt * dlt, axis=0, keepdims=True)
        var = ssq / n_tot
        k = gamma * lax.rsqrt(var + 1e-5)
        return k, beta - mu * k

    riota = lax.broadcasted_iota(jnp.int32, (N, N), 0)
    ciota = lax.broadcasted_iota(jnp.int32, (N, N), 1)
    upmask = jnp.where(ciota > riota, 1.0, 0.0).astype(jnp.float32)

    def tcol(v_row):
        return jnp.transpose(v_row, (1, 0))

    dinvs = []
    for b in range(B):
        raw = adj_ref[b]
        up = jnp.where(raw != 0.0, upmask, 0.0)
        deg = (jnp.sum(up, axis=1, keepdims=True)
               + tcol(jnp.sum(up, axis=0, keepdims=True)) + 1.0)
        dinv_c = lax.rsqrt(deg)
        dinvs.append(dinv_c)
        dinv_r = jnp.transpose(dinv_c, (1, 0))
        t_c = (jnp.sum(up * dinv_r, axis=1, keepdims=True)
               + tcol(jnp.sum(up * dinv_c, axis=0, keepdims=True))
               + dinv_c)
        s_c = dinv_c * t_c
        h0 = jnp.maximum(s_c * W0_ref[...] + b0_ref[...], 0.0)
        out_ref[b * N:(b + 1) * N, :] = h0
    k0, c0 = bn_coeffs(g0_ref[...], be0_ref[...])

    for b in range(B):
        up = jnp.where(adj_ref[b] != 0.0, upmask, 0.0)
        dinv_c = dinvs[b]
        x1 = out_ref[b * N:(b + 1) * N, :] * k0 + c0
        Y = jnp.dot(x1, W1_ref[...])
        V = dinv_c * Y
        U1 = (jnp.dot(up, V)
              + lax.dot_general(up, V, (((0,), (0,)), ((), ())))
              + V)
        h1 = jnp.maximum(dinv_c * U1 + b1_ref[...], 0.0)
        out_ref[b * N:(b + 1) * N, :] = h1
    k1, c1 = bn_coeffs(g1_ref[...], be1_ref[...])

    for b in range(B):
        out_ref[b * N:(b + 1) * N, :] = out_ref[b * N:(b + 1) * N, :] * k1 + c1


_gcn = pl.pallas_call(
    _gcn_body,
    out_shape=jax.ShapeDtypeStruct((B * N, HID), jnp.float32),
)


def kernel(data, W0, b0, g0, beta0, W1, b1, g1, beta1):
    adj = _build_unpack()(data)
    r = lambda v: v.reshape(1, HID)
    return _gcn(adj, W0.reshape(1, HID), r(b0), r(g0), r(beta0),
                W1, r(b1), r(g1), r(beta1))


# speedup vs baseline: 1.1581x; 1.1581x over previous
import functools

import jax
import jax.numpy as jnp
from jax import lax
from jax.experimental import pallas as pl
from jax.experimental.pallas import tpu as pltpu
from jax.experimental.pallas import tpu_sc as plsc

N = 400
B = 16
HID = 16
D = N * (N - 1) // 2
NC, NS = 2, 16
NG = N // 16
NPAIR = (NG - 1) // 2


def _unpack_body(data_hbm, adj_hbm, data_v, rowbuf_v):
    w = lax.axis_index("c")
    b = lax.axis_index("s")
    pltpu.sync_copy(data_hbm.at[b], data_v)
    lanes = lax.iota(jnp.int32, 16)
    jvs = [lanes + cc * 16 for cc in range(NG)]

    def do_group(g):
        base_i = g * 16
        ups = []
        for r in range(16):
            i = base_i + r
            oi = 399 * i - lax.shift_right_logical(i * (i - 1), 1)
            ups.append(oi - i - 1)
        for cc in range(NG):
            @pl.when(cc >= g)
            def _():
                idxs = [jvs[cc] + ups[r] for r in range(16)]
                if cc == 0:
                    idxs = [jnp.maximum(ix, 0) for ix in idxs]
                vals = [plsc.load_gather(data_v, [ix]) for ix in idxs]
                for r in range(16):
                    rowbuf_v[r, pl.ds(cc * 16, 16)] = vals[r]
        pltpu.sync_copy(rowbuf_v, adj_hbm.at[b, pl.ds(base_i, 16)])

    def pair_loop(p, carry):
        ga = p + NPAIR // 2 * w
        do_group(ga)
        do_group(NG - 1 - ga)
        return carry

    lax.fori_loop(0, NPAIR // 2, pair_loop, 0)

    mid_base = (NG // 2) * 16 + 8 * w
    mids = []
    for r in range(8):
        i = mid_base + r
        oi = 399 * i - lax.shift_right_logical(i * (i - 1), 1)
        mids.append(oi - i - 1)
    for cc in range(NG // 2, NG):
        vals = [plsc.load_gather(data_v, [jvs[cc] + mids[r]]) for r in range(8)]
        for r in range(8):
            rowbuf_v[r, pl.ds(cc * 16, 16)] = vals[r]
    pltpu.sync_copy(rowbuf_v.at[pl.ds(0, 8)],
                    adj_hbm.at[b, pl.ds(mid_base, 8)])


@functools.cache
def _build_unpack():
    return pl.kernel(
        _unpack_body,
        out_type=jax.ShapeDtypeStruct((B, N, N), jnp.float32),
        mesh=plsc.VectorSubcoreMesh(
            core_axis_name="c", subcore_axis_name="s",
            num_cores=NC, num_subcores=NS,
        ),
        scratch_types=[
            pltpu.VMEM((D,), jnp.float32),
            pltpu.VMEM((16, N), jnp.float32),
        ],
        compiler_params=pltpu.CompilerParams(needs_layout_passes=False),
    )


def _gcn_body(adj_ref, W0_ref, b0_ref, g0_ref, be0_ref,
              W1_ref, b1_ref, g1_ref, be1_ref, out_ref):
    n_tot = jnp.float32(B * N)

    def bn_coeffs(gamma, beta):
        ssum = jnp.zeros((1, HID), jnp.float32)
        for b in range(B):
            ssum = ssum + jnp.sum(out_ref[b * N:(b + 1) * N, :],
                                  axis=0, keepdims=True)
        mu = ssum / n_tot
        ssq = jnp.zeros((1, HID), jnp.float32)
        for b in range(B):
            dlt = out_ref[b * N:(b + 1) * N, :] - mu
            ssq = ssq + jnp.sum(dlt * dlt, axis=0, keepdims=True)
        var = ssq / n_tot
        k = gamma * lax.rsqrt(var + 1e-5)
        return k, beta - mu * k

    riota = lax.broadcasted_iota(jnp.int32, (N, N), 0)
    ciota = lax.broadcasted_iota(jnp.int32, (N, N), 1)
    upmask = jnp.where(ciota > riota, 1.0, 0.0).astype(jnp.float32)
    eye = jnp.where(ciota == riota, 1.0, 0.0).astype(jnp.float32)

    def build_A(b):
        up = jnp.where(adj_ref[b] != 0.0, upmask, 0.0)
        return up + jnp.transpose(up, (1, 0)) + eye

    dinvs = []
    for b in range(B):
        A = build_A(b)
        deg = jnp.sum(A, axis=1, keepdims=True)
        dinv_c = lax.rsqrt(deg)
        dinvs.append(dinv_c)
        D16 = jnp.broadcast_to(dinv_c, (N, HID))
        M = jnp.dot(A, D16)
        s16 = dinv_c * M
        h0 = jnp.maximum(s16 * W0_ref[...] + b0_ref[...], 0.0)
        out_ref[b * N:(b + 1) * N, :] = h0
    k0, c0 = bn_coeffs(g0_ref[...], be0_ref[...])

    for b in range(B):
        A = build_A(b)
        dinv_c = dinvs[b]
        x1 = out_ref[b * N:(b + 1) * N, :] * k0 + c0
        Y = jnp.dot(x1, W1_ref[...])
        U1 = jnp.dot(A, dinv_c * Y)
        h1 = jnp.maximum(dinv_c * U1 + b1_ref[...], 0.0)
        out_ref[b * N:(b + 1) * N, :] = h1
    k1, c1 = bn_coeffs(g1_ref[...], be1_ref[...])

    for b in range(B):
        out_ref[b * N:(b + 1) * N, :] = out_ref[b * N:(b + 1) * N, :] * k1 + c1


_gcn = pl.pallas_call(
    _gcn_body,
    out_shape=jax.ShapeDtypeStruct((B * N, HID), jnp.float32),
)


def kernel(data, W0, b0, g0, beta0, W1, b1, g1, beta1):
    adj = _build_unpack()(data)
    r = lambda v: v.reshape(1, HID)
    return _gcn(adj, W0.reshape(1, HID), r(b0), r(g0), r(beta0),
                W1, r(b1), r(g1), r(beta1))
